# baseline (speedup 1.0000x reference)
import functools

import jax
import jax.numpy as jnp
from jax import lax
from jax.experimental import pallas as pl
from jax.experimental.pallas import tpu as pltpu
from jax.experimental.pallas import tpu_sc as plsc


def _embed_rows(table, idx):
    V, D = table.shape
    N = idx.shape[0]
    NC, NS = 2, 16
    NW = NC * NS
    bpw = N // NW
    mesh = plsc.VectorSubcoreMesh(core_axis_name="c", subcore_axis_name="s")

    @functools.partial(
        pl.kernel,
        mesh=mesh,
        out_type=jax.ShapeDtypeStruct((N, D), jnp.float32),
        scratch_types=[
            pltpu.VMEM((bpw,), jnp.int32),
            pltpu.VMEM((bpw, D), jnp.float32),
            pltpu.SemaphoreType.DMA,
        ],
    )
    def k(table_hbm, idx_hbm, out_hbm, idx_v, rows_v, sem):
        wid = lax.axis_index("s") * NC + lax.axis_index("c")
        base = wid * bpw
        pltpu.sync_copy(idx_hbm.at[pl.ds(base, bpw)], idx_v)
        pltpu.async_copy(table_hbm.at[idx_v], rows_v, sem).wait()
        pltpu.sync_copy(rows_v, out_hbm.at[pl.ds(base, bpw)])

    return k(table, idx)


def _gru_body(emb_ref, wih_ref, whh_ref, bih_ref, bhh_ref,
              iatW_ref, iatb_ref, gatW_ref, gatb_ref,
              qia_ref, qga_ref, h_scr):
    t = pl.program_id(0)
    nt = pl.num_programs(0)

    @pl.when(t == 0)
    def _():
        h_scr[...] = jnp.zeros_like(h_scr)

    x = emb_ref[0]
    h = h_scr[...]
    H = h.shape[1]
    gi = lax.dot_general(x, wih_ref[...], (((1,), (1,)), ((), ())),
                         preferred_element_type=jnp.float32) + bih_ref[...]
    gh = lax.dot_general(h, whh_ref[...], (((1,), (1,)), ((), ())),
                         preferred_element_type=jnp.float32) + bhh_ref[...]
    r = jax.nn.sigmoid(gi[:, :H] + gh[:, :H])
    z = jax.nn.sigmoid(gi[:, H:2 * H] + gh[:, H:2 * H])
    n = jnp.tanh(gi[:, 2 * H:] + r * gh[:, 2 * H:])
    h_new = (1.0 - z) * n + z * h
    h_scr[...] = h_new

    @pl.when(t == nt - 1)
    def _():
        qia_ref[...] = jax.nn.relu(
            lax.dot_general(h_new, iatW_ref[...], (((1,), (1,)), ((), ())),
                            preferred_element_type=jnp.float32) + iatb_ref[...])
        qga_ref[...] = jax.nn.relu(
            lax.dot_general(h_new, gatW_ref[...], (((1,), (1,)), ((), ())),
                            preferred_element_type=jnp.float32) + gatb_ref[...])


def _gru_encode(emb_tbe, W_ih, W_hh, b_ih, b_hh, ia_txt_W, ia_txt_b,
                ga_txt_W, ga_txt_b):
    T, B, EP = emb_tbe.shape
    H = W_hh.shape[1]
    full = lambda shape: pl.BlockSpec(shape, lambda t: (0,) * len(shape))
    return pl.pallas_call(
        _gru_body,
        grid=(T,),
        in_specs=[
            pl.BlockSpec((1, B, EP), lambda t: (t, 0, 0)),
            full(W_ih.shape), full(W_hh.shape),
            full((1, 3 * H)), full((1, 3 * H)),
            full(ia_txt_W.shape), full((1, H)),
            full(ga_txt_W.shape), full((1, H)),
        ],
        out_specs=[full((B, H)), full((B, H))],
        out_shape=[jax.ShapeDtypeStruct((B, H), jnp.float32)] * 2,
        scratch_shapes=[pltpu.VMEM((B, H), jnp.float32)],
    )(emb_tbe, W_ih, W_hh, b_ih.reshape(1, -1), b_hh.reshape(1, -1),
      ia_txt_W, ia_txt_b.reshape(1, -1), ga_txt_W, ga_txt_b.reshape(1, -1))


_S_ATT = 8
_NBH = 10


def _att_body(img_ref, q_ref, iaW_ref, iab_ref, attW_ref, attb_ref,
              comb_ref, topk_ref):
    S, L, F = img_ref.shape
    R = S * L
    H = q_ref.shape[1]
    X = jnp.reshape(img_ref[...], (R, F))
    PROJ = jax.nn.relu(
        lax.dot_general(X, iaW_ref[...], (((1,), (1,)), ((), ())),
                        preferred_element_type=jnp.float32) + iab_ref[...])
    qv = q_ref[...]
    Q = jnp.reshape(jnp.broadcast_to(qv[:, None, :], (S, L, H)), (R, H))
    joint = PROJ * Q
    jb = joint.astype(jnp.bfloat16).astype(jnp.float32)
    ab = attW_ref[...].astype(jnp.bfloat16).astype(jnp.float32)
    rawc = jnp.sum(jb * ab, axis=1, keepdims=True)
    raw = jnp.reshape(rawc, (S, L)) + attb_ref[0, 0]
    m = jnp.max(raw, axis=1, keepdims=True)
    e = jnp.exp(raw - m)
    att = e / jnp.sum(e, axis=1, keepdims=True)

    ri = lax.broadcasted_iota(jnp.int32, (S, R), 0)
    cj = lax.broadcasted_iota(jnp.int32, (S, R), 1) // L
    ATT = jnp.where(ri == cj, jnp.broadcast_to(jnp.reshape(att, (1, R)),
                                               (S, R)), 0.0)
    IMGATT = lax.dot_general(ATT, X, (((1,), (0,)), ((), ())),
                             preferred_element_type=jnp.float32)
    C = jax.nn.relu(
        lax.dot_general(IMGATT, iaW_ref[...], (((1,), (1,)), ((), ())),
                        preferred_element_type=jnp.float32) + iab_ref[...])
    comb_ref[...] = C * qv

    iota = lax.broadcasted_iota(jnp.int32, (1, L), 1)
    for s in range(S):
        scores = raw[s:s + 1, :]
        for j in range(_NBH):
            mv = jnp.max(scores)
            idx = jnp.min(jnp.where(scores == mv, iota, L))
            topk_ref[s, j:j + 1, :] = img_ref[s, pl.ds(idx, 1), :]
            scores = jnp.where(iota == idx, -jnp.inf, scores)


def _attention1(image, q_ia, ia_img_W, ia_img_b, ia_att_W, ia_att_b):
    B, L, F = image.shape
    H = ia_img_W.shape[0]
    S = _S_ATT
    full = lambda shape: pl.BlockSpec(shape, lambda i: (0,) * len(shape))
    return pl.pallas_call(
        _att_body,
        grid=(B // S,),
        in_specs=[
            pl.BlockSpec((S, L, F), lambda i: (i, 0, 0)),
            pl.BlockSpec((S, H), lambda i: (i, 0)),
            full(ia_img_W.shape), full((1, H)), full((1, H)), full((1, 1)),
        ],
        out_specs=[
            pl.BlockSpec((S, H), lambda i: (i, 0)),
            pl.BlockSpec((S, _NBH, F), lambda i: (i, 0, 0)),
        ],
        out_shape=[
            jax.ShapeDtypeStruct((B, H), jnp.float32),
            jax.ShapeDtypeStruct((B, _NBH, F), jnp.float32),
        ],
    )(image, q_ia, ia_img_W, ia_img_b.reshape(1, -1),
      ia_att_W.reshape(1, -1), ia_att_b.reshape(1, 1))


_S_G = 16
_NK = 8


def _graph_body(ti_ref, q_ref, g1mu_ref, g1sg_ref, g1W_ref, g1b_ref,
                g2mu_ref, g2sg_ref, g2W_ref, g2b_ref,
                gaW_ref, gab_ref, gattW_ref, gattb_ref, out_ref,
                hg1_s, gf_s):
    S, NB, F = ti_ref.shape
    R = S * NB
    X = jnp.reshape(ti_ref[...], (R, F))
    H2 = g1W_ref.shape[2] * _NK
    H = g2W_ref.shape[2] * _NK

    bb = X[:, F - 4:]
    cx = bb[:, 0:1] + 0.5 * (bb[:, 2:3] - bb[:, 0:1])
    cy = bb[:, 1:2] + 0.5 * (bb[:, 3:4] - bb[:, 1:2])
    pcx = cx - jnp.transpose(cx)
    pcy = cy - jnp.transpose(cy)
    rho = jnp.sqrt(pcx * pcx + pcy * pcy)
    theta = jnp.arctan2(pcx, pcy)

    ri = lax.broadcasted_iota(jnp.int32, (R, R), 0) // NB
    ci = lax.broadcasted_iota(jnp.int32, (R, R), 1) // NB
    same = ri == ci

    def w_k(mu_ref, sg_ref, k):
        d0 = (rho - mu_ref[k, 0]) / (1e-14 + sg_ref[k, 0])
        d1 = (theta - mu_ref[k, 1]) / (1e-14 + sg_ref[k, 1])
        return jnp.where(same, jnp.exp(-0.5 * (d0 * d0 + d1 * d1)), 0.0)

    def gconv(mu_ref, sg_ref, W_ref, b_ref, src, dst, dst_off):
        Do = W_ref.shape[2]
        for k in range(_NK):
            agg = lax.dot_general(w_k(mu_ref, sg_ref, k), src,
                                  (((1,), (0,)), ((), ())),
                                  preferred_element_type=jnp.float32)
            o = lax.dot_general(agg, W_ref[k], (((1,), (0,)), ((), ())),
                                preferred_element_type=jnp.float32)
            dst[:, dst_off + k * Do:dst_off + (k + 1) * Do] = jax.nn.relu(
                o + b_ref[:, k * Do:(k + 1) * Do])

    gconv(g1mu_ref, g1sg_ref, g1W_ref, g1b_ref, X, hg1_s, 0)
    HG1 = hg1_s[...]
    gf_s[:, :F] = X
    gconv(g2mu_ref, g2sg_ref, g2W_ref, g2b_ref, HG1, gf_s, F)
    GF = gf_s[...]
    PROJ = jax.nn.relu(
        lax.dot_general(GF, gaW_ref[...], (((1,), (1,)), ((), ())),
                        preferred_element_type=jnp.float32) + gab_ref[...])
    qv = q_ref[...]
    Q = jnp.reshape(jnp.broadcast_to(qv[:, None, :], (S, NB, H)), (R, H))
    rawv = jnp.sum(PROJ * (Q * gattW_ref[...]), axis=1, keepdims=True)
    rawv = rawv + gattb_ref[0, 0]
    Rm = jnp.reshape(rawv, (S, NB))
    m = jnp.max(Rm, axis=1, keepdims=True)
    e = jnp.exp(Rm - m)
    A = e / jnp.sum(e, axis=1, keepdims=True)

    arow = jnp.reshape(A, (1, R))
    si = lax.broadcasted_iota(jnp.int32, (S, R), 0)
    cj = lax.broadcasted_iota(jnp.int32, (S, R), 1) // NB
    ATT = jnp.where(si == cj, jnp.broadcast_to(arow, (S, R)), 0.0)
    ATTD = lax.dot_general(ATT, GF, (((1,), (0,)), ((), ())),
                           preferred_element_type=jnp.float32)
    C = jax.nn.relu(
        lax.dot_general(ATTD, gaW_ref[...], (((1,), (1,)), ((), ())),
                        preferred_element_type=jnp.float32) + gab_ref[...])
    out_ref[...] = C * qv


def _graph_stage(topk_img, q_ga, gc1_mu, gc1_sigma, gc1_W, gc1_b,
                 gc2_mu, gc2_sigma, gc2_W, gc2_b,
                 ga_img_W, ga_img_b, ga_att_W, ga_att_b):
    B, NB, F = topk_img.shape
    H = q_ga.shape[1]
    S = _S_G
    full = lambda shape: pl.BlockSpec(shape, lambda i: (0,) * len(shape))
    smem = lambda shape: pl.BlockSpec(memory_space=pltpu.SMEM)
    return pl.pallas_call(
        _graph_body,
        grid=(B // S,),
        in_specs=[
            pl.BlockSpec((S, NB, F), lambda i: (i, 0, 0)),
            pl.BlockSpec((S, H), lambda i: (i, 0)),
            smem(gc1_mu.shape), smem(gc1_sigma.shape),
            full(gc1_W.shape), full((1, 2 * H)),
            smem(gc2_mu.shape), smem(gc2_sigma.shape),
            full(gc2_W.shape), full((1, H)),
            full(ga_img_W.shape), full((1, H)), full((1, H)), full((1, 1)),
        ],
        out_specs=pl.BlockSpec((S, H), lambda i: (i, 0)),
        out_shape=jax.ShapeDtypeStruct((B, H), jnp.float32),
        scratch_shapes=[
            pltpu.VMEM((S * NB, 2 * H), jnp.float32),
            pltpu.VMEM((S * NB, F + H), jnp.float32),
        ],
    )(topk_img, q_ga, gc1_mu, gc1_sigma, gc1_W, gc1_b.reshape(1, -1),
      gc2_mu, gc2_sigma, gc2_W, gc2_b.reshape(1, -1),
      ga_img_W, ga_img_b.reshape(1, -1), ga_att_W.reshape(1, -1),
      ga_att_b.reshape(1, 1))


_T_OUT = 512


def _head1_body(c2_ref, c1_ref, o1W_ref, o1b_ref, io1W_ref, io1b_ref,
                h1_ref, h2_ref):
    h1_ref[...] = jax.nn.relu(
        lax.dot_general(c2_ref[...], o1W_ref[...], (((1,), (1,)), ((), ())),
                        preferred_element_type=jnp.float32) + o1b_ref[...])
    h2_ref[...] = jax.nn.relu(
        lax.dot_general(c1_ref[...], io1W_ref[...], (((1,), (1,)), ((), ())),
                        preferred_element_type=jnp.float32) + io1b_ref[...])


def _head2_body(h1_ref, h2_ref, o2W_ref, o2b_ref, io2W_ref, io2b_ref,
                out_ref):
    out_ref[...] = (
        lax.dot_general(h1_ref[...], o2W_ref[...], (((1,), (1,)), ((), ())),
                        preferred_element_type=jnp.float32)
        + lax.dot_general(h2_ref[...], io2W_ref[...], (((1,), (1,)), ((), ())),
                          preferred_element_type=jnp.float32)
        + o2b_ref[...] + io2b_ref[...])


def _heads(comb2, comb1, out1_W, out1_b, out2_W, out2_b,
           iout1_W, iout1_b, iout2_W, iout2_b):
    B, H = comb2.shape
    O = out1_W.shape[0]
    T = _T_OUT
    nj = pl.cdiv(O, T)
    full = lambda shape: pl.BlockSpec(shape, lambda j: (0,) * len(shape))
    h1, h2 = pl.pallas_call(
        _head1_body,
        grid=(nj,),
        in_specs=[
            full((B, H)), full((B, H)),
            pl.BlockSpec((T, H), lambda j: (j, 0)),
            pl.BlockSpec((1, T), lambda j: (0, j)),
            pl.BlockSpec((T, H), lambda j: (j, 0)),
            pl.BlockSpec((1, T), lambda j: (0, j)),
        ],
        out_specs=[pl.BlockSpec((B, T), lambda j: (0, j))] * 2,
        out_shape=[jax.ShapeDtypeStruct((B, O), jnp.float32)] * 2,
    )(comb2, comb1, out1_W, out1_b.reshape(1, -1),
      iout1_W, iout1_b.reshape(1, -1))

    return pl.pallas_call(
        _head2_body,
        grid=(nj,),
        in_specs=[
            full((B, O)), full((B, O)),
            pl.BlockSpec((T, O), lambda j: (j, 0)),
            pl.BlockSpec((1, T), lambda j: (0, j)),
            pl.BlockSpec((T, O), lambda j: (j, 0)),
            pl.BlockSpec((1, T), lambda j: (0, j)),
        ],
        out_specs=pl.BlockSpec((B, T), lambda j: (0, j)),
        out_shape=jax.ShapeDtypeStruct((B, O), jnp.float32),
    )(h1, h2, out2_W, out2_b.reshape(1, -1), iout2_W, iout2_b.reshape(1, -1))


def kernel(question, image, K, qlen, wembed, W_ih, W_hh, b_ih, b_hh,
           ia_img_W, ia_img_b, ia_txt_W, ia_txt_b, ia_att_W, ia_att_b,
           ga_img_W, ga_img_b, ga_txt_W, ga_txt_b, ga_att_W, ga_att_b,
           gc1_mu, gc1_sigma, gc1_W, gc1_b, gc2_mu, gc2_sigma, gc2_W, gc2_b,
           out1_W, out1_b, out2_W, out2_b, iout1_W, iout1_b, iout2_W, iout2_b):
    B, T = question.shape
    V, E = wembed.shape

    EP = ((E + 127) // 128) * 128
    table = jnp.pad(wembed, ((0, 0), (0, EP - E))) if EP != E else wembed
    idx = question.T.reshape(-1).astype(jnp.int32)
    emb = _embed_rows(table, idx).reshape(T, B, EP)

    W_ih_p = jnp.pad(W_ih, ((0, 0), (0, EP - E))) if EP != E else W_ih
    q_ia, q_ga = _gru_encode(emb, W_ih_p, W_hh, b_ih, b_hh,
                             ia_txt_W, ia_txt_b, ga_txt_W, ga_txt_b)

    comb1, topk_img = _attention1(image, q_ia, ia_img_W, ia_img_b,
                                  ia_att_W, ia_att_b)

    comb2 = _graph_stage(topk_img, q_ga, gc1_mu, gc1_sigma, gc1_W, gc1_b,
                         gc2_mu, gc2_sigma, gc2_W, gc2_b,
                         ga_img_W, ga_img_b, ga_att_W, ga_att_b)

    return _heads(comb2, comb1, out1_W, out1_b, out2_W, out2_b,
                  iout1_W, iout1_b, iout2_W, iout2_b)


# speedup vs baseline: 1.3041x; 1.3041x over previous
import functools

import jax
import jax.numpy as jnp
from jax import lax
from jax.experimental import pallas as pl
from jax.experimental.pallas import tpu as pltpu
from jax.experimental.pallas import tpu_sc as plsc


def _embed_rows(table, idx):
    V, D = table.shape
    N = idx.shape[0]
    NC, NS = 2, 16
    NW = NC * NS
    bpw = N // NW
    mesh = plsc.VectorSubcoreMesh(core_axis_name="c", subcore_axis_name="s")

    @functools.partial(
        pl.kernel,
        mesh=mesh,
        out_type=jax.ShapeDtypeStruct((N, D), jnp.float32),
        scratch_types=[
            pltpu.VMEM((bpw,), jnp.int32),
            pltpu.VMEM((bpw, D), jnp.float32),
            pltpu.SemaphoreType.DMA,
        ],
    )
    def k(table_hbm, idx_hbm, out_hbm, idx_v, rows_v, sem):
        wid = lax.axis_index("s") * NC + lax.axis_index("c")
        base = wid * bpw
        pltpu.sync_copy(idx_hbm.at[pl.ds(base, bpw)], idx_v)
        pltpu.async_copy(table_hbm.at[idx_v], rows_v, sem).wait()
        pltpu.sync_copy(rows_v, out_hbm.at[pl.ds(base, bpw)])

    return k(table, idx)


def _pad_body(src_ref, dst_ref):
    E = src_ref.shape[1]
    dst_ref[...] = jnp.zeros_like(dst_ref)
    dst_ref[:, :E] = src_ref[...]


def _pad_cols(src, EP):
    V, E = src.shape
    TV = 2000
    return pl.pallas_call(
        _pad_body,
        grid=(V // TV,),
        in_specs=[pl.BlockSpec((TV, E), lambda i: (i, 0))],
        out_specs=pl.BlockSpec((TV, EP), lambda i: (i, 0)),
        out_shape=jax.ShapeDtypeStruct((V, EP), jnp.float32),
    )(src)


def _gru_body(emb_ref, wih_ref, whh_ref, bih_ref, bhh_ref,
              iatW_ref, iatb_ref, gatW_ref, gatb_ref,
              qia_ref, qga_ref, h_scr):
    t = pl.program_id(0)
    nt = pl.num_programs(0)

    @pl.when(t == 0)
    def _():
        h_scr[...] = jnp.zeros_like(h_scr)

    x = emb_ref[0]
    h = h_scr[...]
    H = h.shape[1]
    gi = lax.dot_general(x, wih_ref[...], (((1,), (1,)), ((), ())),
                         preferred_element_type=jnp.float32) + bih_ref[...]
    gh = lax.dot_general(h, whh_ref[...], (((1,), (1,)), ((), ())),
                         preferred_element_type=jnp.float32) + bhh_ref[...]
    r = jax.nn.sigmoid(gi[:, :H] + gh[:, :H])
    z = jax.nn.sigmoid(gi[:, H:2 * H] + gh[:, H:2 * H])
    n = jnp.tanh(gi[:, 2 * H:] + r * gh[:, 2 * H:])
    h_new = (1.0 - z) * n + z * h
    h_scr[...] = h_new

    @pl.when(t == nt - 1)
    def _():
        qia_ref[...] = jax.nn.relu(
            lax.dot_general(h_new, iatW_ref[...], (((1,), (1,)), ((), ())),
                            preferred_element_type=jnp.float32) + iatb_ref[...])
        qga_ref[...] = jax.nn.relu(
            lax.dot_general(h_new, gatW_ref[...], (((1,), (1,)), ((), ())),
                            preferred_element_type=jnp.float32) + gatb_ref[...])


def _gru_encode(emb_tbe, W_ih, W_hh, b_ih, b_hh, ia_txt_W, ia_txt_b,
                ga_txt_W, ga_txt_b):
    T, B, EP = emb_tbe.shape
    H = W_hh.shape[1]
    full = lambda shape: pl.BlockSpec(shape, lambda t: (0,) * len(shape))
    return pl.pallas_call(
        _gru_body,
        grid=(T,),
        in_specs=[
            pl.BlockSpec((1, B, EP), lambda t: (t, 0, 0)),
            full(W_ih.shape), full(W_hh.shape),
            full((1, 3 * H)), full((1, 3 * H)),
            full(ia_txt_W.shape), full((1, H)),
            full(ga_txt_W.shape), full((1, H)),
        ],
        out_specs=[full((B, H)), full((B, H))],
        out_shape=[jax.ShapeDtypeStruct((B, H), jnp.float32)] * 2,
        scratch_shapes=[pltpu.VMEM((B, H), jnp.float32)],
    )(emb_tbe, W_ih, W_hh, b_ih.reshape(1, -1), b_hh.reshape(1, -1),
      ia_txt_W, ia_txt_b.reshape(1, -1), ga_txt_W, ga_txt_b.reshape(1, -1))


_S_ATT = 4
_NBH = 10


def _att_body(img_ref, q_ref, iaW_ref, iab_ref, attW_ref, attb_ref,
              comb_ref, topk_ref):
    S, L, F = img_ref.shape
    R = S * L
    H = q_ref.shape[2]
    X = jnp.reshape(img_ref[...], (R, F))
    PROJ = jax.nn.relu(
        lax.dot_general(X, iaW_ref[...], (((1,), (1,)), ((), ())),
                        preferred_element_type=jnp.float32) + iab_ref[...])
    qv = q_ref[0]
    Q = jnp.reshape(jnp.broadcast_to(qv[:, None, :], (S, L, H)), (R, H))
    joint = PROJ * Q
    jb = joint.astype(jnp.bfloat16).astype(jnp.float32)
    ab = attW_ref[...].astype(jnp.bfloat16).astype(jnp.float32)
    rawc = jnp.sum(jb * ab, axis=1, keepdims=True)
    raw = jnp.reshape(rawc, (S, L)) + attb_ref[0, 0]
    m = jnp.max(raw, axis=1, keepdims=True)
    e = jnp.exp(raw - m)
    att = e / jnp.sum(e, axis=1, keepdims=True)

    ri = lax.broadcasted_iota(jnp.int32, (S, R), 0)
    cj = lax.broadcasted_iota(jnp.int32, (S, R), 1) // L
    ATT = jnp.where(ri == cj, jnp.broadcast_to(jnp.reshape(att, (1, R)),
                                               (S, R)), 0.0)
    IMGATT = lax.dot_general(ATT, X, (((1,), (0,)), ((), ())),
                             preferred_element_type=jnp.float32)
    C = jax.nn.relu(
        lax.dot_general(IMGATT, iaW_ref[...], (((1,), (1,)), ((), ())),
                        preferred_element_type=jnp.float32) + iab_ref[...])
    comb_ref[0] = C * qv

    G = S * _NBH
    scoresG = jnp.reshape(jnp.broadcast_to(raw[:, None, :], (S, _NBH, L)),
                          (G, L))
    iotaG = lax.broadcasted_iota(jnp.int32, (G, L), 1)
    rq = lax.broadcasted_iota(jnp.int32, (G, 1), 0)
    sG = rq // _NBH
    jG = rq % _NBH
    tgt = jnp.zeros((G, 1), jnp.int32)
    for j in range(_NBH):
        mv = jnp.max(scoresG, axis=1, keepdims=True)
        idx = jnp.min(jnp.where(scoresG == mv, iotaG, L),
                      axis=1, keepdims=True)
        tgt = jnp.where(jG == j, idx + L * sG, tgt)
        scoresG = jnp.where(iotaG == idx, -jnp.inf, scoresG)

    colI = lax.broadcasted_iota(jnp.int32, (G, R), 1)
    SEL = (colI == tgt).astype(jnp.float32)
    TK = lax.dot_general(SEL, X, (((1,), (0,)), ((), ())),
                         preferred_element_type=jnp.float32,
                         precision=lax.Precision.HIGHEST)
    topk_ref[...] = jnp.reshape(TK, (S, _NBH, F))


def _attention1(image, q_ia, ia_img_W, ia_img_b, ia_att_W, ia_att_b):
    B, L, F = image.shape
    H = ia_img_W.shape[0]
    S = _S_ATT
    full = lambda shape: pl.BlockSpec(shape, lambda i: (0,) * len(shape))
    comb3, topk = pl.pallas_call(
        _att_body,
        grid=(B // S,),
        in_specs=[
            pl.BlockSpec((S, L, F), lambda i: (i, 0, 0)),
            pl.BlockSpec((1, S, H), lambda i: (i, 0, 0)),
            full(ia_img_W.shape), full((1, H)), full((1, H)), full((1, 1)),
        ],
        out_specs=[
            pl.BlockSpec((1, S, H), lambda i: (i, 0, 0)),
            pl.BlockSpec((S, _NBH, F), lambda i: (i, 0, 0)),
        ],
        out_shape=[
            jax.ShapeDtypeStruct((B // S, S, H), jnp.float32),
            jax.ShapeDtypeStruct((B, _NBH, F), jnp.float32),
        ],
    )(image, q_ia.reshape(B // S, S, H), ia_img_W, ia_img_b.reshape(1, -1),
      ia_att_W.reshape(1, -1), ia_att_b.reshape(1, 1))
    return comb3.reshape(B, H), topk


_S_G = 16
_NK = 8


def _graph_body(ti_ref, q_ref, g1mu_ref, g1sg_ref, g1W_ref, g1b_ref,
                g2mu_ref, g2sg_ref, g2W_ref, g2b_ref,
                gaW_ref, gab_ref, gattW_ref, gattb_ref, out_ref,
                hg1_s, gf_s):
    S, NB, F = ti_ref.shape
    R = S * NB
    X = jnp.reshape(ti_ref[...], (R, F))
    H2 = g1W_ref.shape[2] * _NK
    H = g2W_ref.shape[2] * _NK

    bb = X[:, F - 4:]
    cx = bb[:, 0:1] + 0.5 * (bb[:, 2:3] - bb[:, 0:1])
    cy = bb[:, 1:2] + 0.5 * (bb[:, 3:4] - bb[:, 1:2])
    pcx = cx - jnp.transpose(cx)
    pcy = cy - jnp.transpose(cy)
    rho = jnp.sqrt(pcx * pcx + pcy * pcy)
    theta = jnp.arctan2(pcx, pcy)

    ri = lax.broadcasted_iota(jnp.int32, (R, R), 0) // NB
    ci = lax.broadcasted_iota(jnp.int32, (R, R), 1) // NB
    same = ri == ci

    def w_k(mu_ref, sg_ref, k):
        d0 = (rho - mu_ref[k, 0]) / (1e-14 + sg_ref[k, 0])
        d1 = (theta - mu_ref[k, 1]) / (1e-14 + sg_ref[k, 1])
        return jnp.where(same, jnp.exp(-0.5 * (d0 * d0 + d1 * d1)), 0.0)

    def gconv(mu_ref, sg_ref, W_ref, b_ref, src, dst, dst_off):
        Do = W_ref.shape[2]
        for k in range(_NK):
            agg = lax.dot_general(w_k(mu_ref, sg_ref, k), src,
                                  (((1,), (0,)), ((), ())),
                                  preferred_element_type=jnp.float32)
            o = lax.dot_general(agg, W_ref[k], (((1,), (0,)), ((), ())),
                                preferred_element_type=jnp.float32)
            dst[:, dst_off + k * Do:dst_off + (k + 1) * Do] = jax.nn.relu(
                o + b_ref[:, k * Do:(k + 1) * Do])

    gconv(g1mu_ref, g1sg_ref, g1W_ref, g1b_ref, X, hg1_s, 0)
    HG1 = hg1_s[...]
    gf_s[:, :F] = X
    gconv(g2mu_ref, g2sg_ref, g2W_ref, g2b_ref, HG1, gf_s, F)
    GF = gf_s[...]
    PROJ = jax.nn.relu(
        lax.dot_general(GF, gaW_ref[...], (((1,), (1,)), ((), ())),
                        preferred_element_type=jnp.float32) + gab_ref[...])
    qv = q_ref[...]
    Q = jnp.reshape(jnp.broadcast_to(qv[:, None, :], (S, NB, H)), (R, H))
    rawv = jnp.sum(PROJ * (Q * gattW_ref[...]), axis=1, keepdims=True)
    rawv = rawv + gattb_ref[0, 0]
    Rm = jnp.reshape(rawv, (S, NB))
    m = jnp.max(Rm, axis=1, keepdims=True)
    e = jnp.exp(Rm - m)
    A = e / jnp.sum(e, axis=1, keepdims=True)

    arow = jnp.reshape(A, (1, R))
    si = lax.broadcasted_iota(jnp.int32, (S, R), 0)
    cj = lax.broadcasted_iota(jnp.int32, (S, R), 1) // NB
    ATT = jnp.where(si == cj, jnp.broadcast_to(arow, (S, R)), 0.0)
    ATTD = lax.dot_general(ATT, GF, (((1,), (0,)), ((), ())),
                           preferred_element_type=jnp.float32)
    C = jax.nn.relu(
        lax.dot_general(ATTD, gaW_ref[...], (((1,), (1,)), ((), ())),
                        preferred_element_type=jnp.float32) + gab_ref[...])
    out_ref[...] = C * qv


def _graph_stage(topk_img, q_ga, gc1_mu, gc1_sigma, gc1_W, gc1_b,
                 gc2_mu, gc2_sigma, gc2_W, gc2_b,
                 ga_img_W, ga_img_b, ga_att_W, ga_att_b):
    B, NB, F = topk_img.shape
    H = q_ga.shape[1]
    S = _S_G
    full = lambda shape: pl.BlockSpec(shape, lambda i: (0,) * len(shape))
    smem = lambda shape: pl.BlockSpec(memory_space=pltpu.SMEM)
    return pl.pallas_call(
        _graph_body,
        grid=(B // S,),
        in_specs=[
            pl.BlockSpec((S, NB, F), lambda i: (i, 0, 0)),
            pl.BlockSpec((S, H), lambda i: (i, 0)),
            smem(gc1_mu.shape), smem(gc1_sigma.shape),
            full(gc1_W.shape), full((1, 2 * H)),
            smem(gc2_mu.shape), smem(gc2_sigma.shape),
            full(gc2_W.shape), full((1, H)),
            full(ga_img_W.shape), full((1, H)), full((1, H)), full((1, 1)),
        ],
        out_specs=pl.BlockSpec((S, H), lambda i: (i, 0)),
        out_shape=jax.ShapeDtypeStruct((B, H), jnp.float32),
        scratch_shapes=[
            pltpu.VMEM((S * NB, 2 * H), jnp.float32),
            pltpu.VMEM((S * NB, F + H), jnp.float32),
        ],
    )(topk_img, q_ga, gc1_mu, gc1_sigma, gc1_W, gc1_b.reshape(1, -1),
      gc2_mu, gc2_sigma, gc2_W, gc2_b.reshape(1, -1),
      ga_img_W, ga_img_b.reshape(1, -1), ga_att_W.reshape(1, -1),
      ga_att_b.reshape(1, 1))


_T_OUT = 512


def _head1_body(c2_ref, c1_ref, o1W_ref, o1b_ref, io1W_ref, io1b_ref,
                h1_ref, h2_ref):
    h1_ref[...] = jax.nn.relu(
        lax.dot_general(c2_ref[...], o1W_ref[...], (((1,), (1,)), ((), ())),
                        preferred_element_type=jnp.float32) + o1b_ref[...])
    h2_ref[...] = jax.nn.relu(
        lax.dot_general(c1_ref[...], io1W_ref[...], (((1,), (1,)), ((), ())),
                        preferred_element_type=jnp.float32) + io1b_ref[...])


def _head2_body(h1_ref, h2_ref, o2W_ref, o2b_ref, io2W_ref, io2b_ref,
                out_ref):
    out_ref[...] = (
        lax.dot_general(h1_ref[...], o2W_ref[...], (((1,), (1,)), ((), ())),
                        preferred_element_type=jnp.float32)
        + lax.dot_general(h2_ref[...], io2W_ref[...], (((1,), (1,)), ((), ())),
                          preferred_element_type=jnp.float32)
        + o2b_ref[...] + io2b_ref[...])


def _heads(comb2, comb1, out1_W, out1_b, out2_W, out2_b,
           iout1_W, iout1_b, iout2_W, iout2_b):
    B, H = comb2.shape
    O = out1_W.shape[0]
    T = _T_OUT
    nj = pl.cdiv(O, T)
    full = lambda shape: pl.BlockSpec(shape, lambda j: (0,) * len(shape))
    h1, h2 = pl.pallas_call(
        _head1_body,
        grid=(nj,),
        in_specs=[
            full((B, H)), full((B, H)),
            pl.BlockSpec((T, H), lambda j: (j, 0)),
            pl.BlockSpec((1, T), lambda j: (0, j)),
            pl.BlockSpec((T, H), lambda j: (j, 0)),
            pl.BlockSpec((1, T), lambda j: (0, j)),
        ],
        out_specs=[pl.BlockSpec((B, T), lambda j: (0, j))] * 2,
        out_shape=[jax.ShapeDtypeStruct((B, O), jnp.float32)] * 2,
    )(comb2, comb1, out1_W, out1_b.reshape(1, -1),
      iout1_W, iout1_b.reshape(1, -1))

    return pl.pallas_call(
        _head2_body,
        grid=(nj,),
        in_specs=[
            full((B, O)), full((B, O)),
            pl.BlockSpec((T, O), lambda j: (j, 0)),
            pl.BlockSpec((1, T), lambda j: (0, j)),
            pl.BlockSpec((T, O), lambda j: (j, 0)),
            pl.BlockSpec((1, T), lambda j: (0, j)),
        ],
        out_specs=pl.BlockSpec((B, T), lambda j: (0, j)),
        out_shape=jax.ShapeDtypeStruct((B, O), jnp.float32),
    )(h1, h2, out2_W, out2_b.reshape(1, -1), iout2_W, iout2_b.reshape(1, -1))


def kernel(question, image, K, qlen, wembed, W_ih, W_hh, b_ih, b_hh,
           ia_img_W, ia_img_b, ia_txt_W, ia_txt_b, ia_att_W, ia_att_b,
           ga_img_W, ga_img_b, ga_txt_W, ga_txt_b, ga_att_W, ga_att_b,
           gc1_mu, gc1_sigma, gc1_W, gc1_b, gc2_mu, gc2_sigma, gc2_W, gc2_b,
           out1_W, out1_b, out2_W, out2_b, iout1_W, iout1_b, iout2_W, iout2_b):
    B, T = question.shape
    V, E = wembed.shape

    EP = ((E + 127) // 128) * 128
    table = _pad_cols(wembed, EP) if EP != E else wembed
    idx = question.T.reshape(-1).astype(jnp.int32)
    emb = _embed_rows(table, idx).reshape(T, B, EP)

    W_ih_p = jnp.pad(W_ih, ((0, 0), (0, EP - E))) if EP != E else W_ih
    q_ia, q_ga = _gru_encode(emb, W_ih_p, W_hh, b_ih, b_hh,
                             ia_txt_W, ia_txt_b, ga_txt_W, ga_txt_b)

    comb1, topk_img = _attention1(image, q_ia, ia_img_W, ia_img_b,
                                  ia_att_W, ia_att_b)

    comb2 = _graph_stage(topk_img, q_ga, gc1_mu, gc1_sigma, gc1_W, gc1_b,
                         gc2_mu, gc2_sigma, gc2_W, gc2_b,
                         ga_img_W, ga_img_b, ga_att_W, ga_att_b)

    return _heads(comb2, comb1, out1_W, out1_b, out2_W, out2_b,
                  iout1_W, iout1_b, iout2_W, iout2_b)


# speedup vs baseline: 1.3240x; 1.0153x over previous
import functools

import jax
import jax.numpy as jnp
from jax import lax
from jax.experimental import pallas as pl
from jax.experimental.pallas import tpu as pltpu
from jax.experimental.pallas import tpu_sc as plsc


def _embed_rows(table, idx):
    V, D = table.shape
    N = idx.shape[0]
    NC, NS = 2, 16
    NW = NC * NS
    bpw = N // NW
    mesh = plsc.VectorSubcoreMesh(core_axis_name="c", subcore_axis_name="s")

    @functools.partial(
        pl.kernel,
        mesh=mesh,
        out_type=jax.ShapeDtypeStruct((N, D), jnp.float32),
        scratch_types=[
            pltpu.VMEM((bpw,), jnp.int32),
            pltpu.VMEM((bpw, D), jnp.float32),
            pltpu.SemaphoreType.DMA,
        ],
    )
    def k(table_hbm, idx_hbm, out_hbm, idx_v, rows_v, sem):
        wid = lax.axis_index("s") * NC + lax.axis_index("c")
        base = wid * bpw
        pltpu.sync_copy(idx_hbm.at[pl.ds(base, bpw)], idx_v)
        pltpu.async_copy(table_hbm.at[idx_v], rows_v, sem).wait()
        pltpu.sync_copy(rows_v, out_hbm.at[pl.ds(base, bpw)])

    return k(table, idx)


def _pad_body(src_ref, dst_ref):
    E = src_ref.shape[1]
    dst_ref[...] = jnp.zeros_like(dst_ref)
    dst_ref[:, :E] = src_ref[...]


def _pad_cols(src, EP):
    V, E = src.shape
    TV = 2000
    return pl.pallas_call(
        _pad_body,
        grid=(V // TV,),
        in_specs=[pl.BlockSpec((TV, E), lambda i: (i, 0))],
        out_specs=pl.BlockSpec((TV, EP), lambda i: (i, 0)),
        out_shape=jax.ShapeDtypeStruct((V, EP), jnp.float32),
    )(src)


def _gru_body(emb_ref, wih_ref, whh_ref, bih_ref, bhh_ref,
              iatW_ref, iatb_ref, gatW_ref, gatb_ref,
              qia_ref, qga_ref, h_scr):
    t = pl.program_id(0)
    nt = pl.num_programs(0)

    @pl.when(t == 0)
    def _():
        h_scr[...] = jnp.zeros_like(h_scr)

    x = emb_ref[0]
    h = h_scr[...]
    H = h.shape[1]
    gi = lax.dot_general(x, wih_ref[...], (((1,), (1,)), ((), ())),
                         preferred_element_type=jnp.float32) + bih_ref[...]
    gh = lax.dot_general(h, whh_ref[...], (((1,), (1,)), ((), ())),
                         preferred_element_type=jnp.float32) + bhh_ref[...]
    r = jax.nn.sigmoid(gi[:, :H] + gh[:, :H])
    z = jax.nn.sigmoid(gi[:, H:2 * H] + gh[:, H:2 * H])
    n = jnp.tanh(gi[:, 2 * H:] + r * gh[:, 2 * H:])
    h_new = (1.0 - z) * n + z * h
    h_scr[...] = h_new

    @pl.when(t == nt - 1)
    def _():
        qia_ref[...] = jax.nn.relu(
            lax.dot_general(h_new, iatW_ref[...], (((1,), (1,)), ((), ())),
                            preferred_element_type=jnp.float32) + iatb_ref[...])
        qga_ref[...] = jax.nn.relu(
            lax.dot_general(h_new, gatW_ref[...], (((1,), (1,)), ((), ())),
                            preferred_element_type=jnp.float32) + gatb_ref[...])


def _gru_encode(emb_tbe, W_ih, W_hh, b_ih, b_hh, ia_txt_W, ia_txt_b,
                ga_txt_W, ga_txt_b):
    T, B, EP = emb_tbe.shape
    H = W_hh.shape[1]
    full = lambda shape: pl.BlockSpec(shape, lambda t: (0,) * len(shape))
    return pl.pallas_call(
        _gru_body,
        grid=(T,),
        in_specs=[
            pl.BlockSpec((1, B, EP), lambda t: (t, 0, 0)),
            full(W_ih.shape), full(W_hh.shape),
            full((1, 3 * H)), full((1, 3 * H)),
            full(ia_txt_W.shape), full((1, H)),
            full(ga_txt_W.shape), full((1, H)),
        ],
        out_specs=[full((B, H)), full((B, H))],
        out_shape=[jax.ShapeDtypeStruct((B, H), jnp.float32)] * 2,
        scratch_shapes=[pltpu.VMEM((B, H), jnp.float32)],
    )(emb_tbe, W_ih, W_hh, b_ih.reshape(1, -1), b_hh.reshape(1, -1),
      ia_txt_W, ia_txt_b.reshape(1, -1), ga_txt_W, ga_txt_b.reshape(1, -1))


_S_ATT = 4
_NBH = 10


def _att_body(img_ref, q_ref, iaW_ref, iab_ref, attW_ref, attb_ref,
              comb_ref, topk_ref):
    S, L, F = img_ref.shape
    R = S * L
    H = q_ref.shape[2]
    X = jnp.reshape(img_ref[...], (R, F))
    PROJ = jax.nn.relu(
        lax.dot_general(X, iaW_ref[...], (((1,), (1,)), ((), ())),
                        preferred_element_type=jnp.float32) + iab_ref[...])
    qv = q_ref[0]
    Q = jnp.reshape(jnp.broadcast_to(qv[:, None, :], (S, L, H)), (R, H))
    joint = PROJ * Q
    jb = joint.astype(jnp.bfloat16).astype(jnp.float32)
    ab = attW_ref[...].astype(jnp.bfloat16).astype(jnp.float32)
    rawc = jnp.sum(jb * ab, axis=1, keepdims=True)
    raw = jnp.reshape(rawc, (S, L)) + attb_ref[0, 0]
    m = jnp.max(raw, axis=1, keepdims=True)
    e = jnp.exp(raw - m)
    att = e / jnp.sum(e, axis=1, keepdims=True)

    ri = lax.broadcasted_iota(jnp.int32, (S, R), 0)
    cj = lax.broadcasted_iota(jnp.int32, (S, R), 1) // L
    ATT = jnp.where(ri == cj, jnp.broadcast_to(jnp.reshape(att, (1, R)),
                                               (S, R)), 0.0)
    IMGATT = lax.dot_general(ATT, X, (((1,), (0,)), ((), ())),
                             preferred_element_type=jnp.float32)
    C = jax.nn.relu(
        lax.dot_general(IMGATT, iaW_ref[...], (((1,), (1,)), ((), ())),
                        preferred_element_type=jnp.float32) + iab_ref[...])
    comb_ref[0] = C * qv

    G = S * _NBH
    scoresG = jnp.reshape(jnp.broadcast_to(raw[:, None, :], (S, _NBH, L)),
                          (G, L))
    iotaG = lax.broadcasted_iota(jnp.int32, (G, L), 1)
    rq = lax.broadcasted_iota(jnp.int32, (G, 1), 0)
    sG = rq // _NBH
    jG = rq % _NBH
    tgt = jnp.zeros((G, 1), jnp.int32)
    for j in range(_NBH):
        mv = jnp.max(scoresG, axis=1, keepdims=True)
        idx = jnp.min(jnp.where(scoresG == mv, iotaG, L),
                      axis=1, keepdims=True)
        tgt = jnp.where(jG == j, idx + L * sG, tgt)
        scoresG = jnp.where(iotaG == idx, -jnp.inf, scoresG)

    colI = lax.broadcasted_iota(jnp.int32, (G, R), 1)
    SEL = (colI == tgt).astype(jnp.bfloat16)
    hi = X.astype(jnp.bfloat16)
    mid = (X - hi.astype(jnp.float32)).astype(jnp.bfloat16)
    lo = (X - hi.astype(jnp.float32)
          - mid.astype(jnp.float32)).astype(jnp.bfloat16)
    dn = (((1,), (0,)), ((), ()))
    TK = (lax.dot_general(SEL, hi, dn, preferred_element_type=jnp.float32)
          + lax.dot_general(SEL, mid, dn, preferred_element_type=jnp.float32)
          + lax.dot_general(SEL, lo, dn, preferred_element_type=jnp.float32))
    topk_ref[...] = jnp.reshape(TK, (S, _NBH, F))


def _attention1(image, q_ia, ia_img_W, ia_img_b, ia_att_W, ia_att_b):
    B, L, F = image.shape
    H = ia_img_W.shape[0]
    S = _S_ATT
    full = lambda shape: pl.BlockSpec(shape, lambda i: (0,) * len(shape))
    comb3, topk = pl.pallas_call(
        _att_body,
        grid=(B // S,),
        in_specs=[
            pl.BlockSpec((S, L, F), lambda i: (i, 0, 0)),
            pl.BlockSpec((1, S, H), lambda i: (i, 0, 0)),
            full(ia_img_W.shape), full((1, H)), full((1, H)), full((1, 1)),
        ],
        out_specs=[
            pl.BlockSpec((1, S, H), lambda i: (i, 0, 0)),
            pl.BlockSpec((S, _NBH, F), lambda i: (i, 0, 0)),
        ],
        out_shape=[
            jax.ShapeDtypeStruct((B // S, S, H), jnp.float32),
            jax.ShapeDtypeStruct((B, _NBH, F), jnp.float32),
        ],
    )(image, q_ia.reshape(B // S, S, H), ia_img_W, ia_img_b.reshape(1, -1),
      ia_att_W.reshape(1, -1), ia_att_b.reshape(1, 1))
    return comb3.reshape(B, H), topk


_S_G = 16
_NK = 8


def _graph_body(ti_ref, q_ref, g1mu_ref, g1sg_ref, g1W_ref, g1b_ref,
                g2mu_ref, g2sg_ref, g2W_ref, g2b_ref,
                gaW_ref, gab_ref, gattW_ref, gattb_ref, out_ref,
                hg1_s, gf_s):
    S, NB, F = ti_ref.shape
    R = S * NB
    X = jnp.reshape(ti_ref[...], (R, F))
    H2 = g1W_ref.shape[2] * _NK
    H = g2W_ref.shape[2] * _NK

    bb = X[:, F - 4:]
    cx = bb[:, 0:1] + 0.5 * (bb[:, 2:3] - bb[:, 0:1])
    cy = bb[:, 1:2] + 0.5 * (bb[:, 3:4] - bb[:, 1:2])
    pcx = cx - jnp.transpose(cx)
    pcy = cy - jnp.transpose(cy)
    rho = jnp.sqrt(pcx * pcx + pcy * pcy)
    theta = jnp.arctan2(pcx, pcy)

    ri = lax.broadcasted_iota(jnp.int32, (R, R), 0) // NB
    ci = lax.broadcasted_iota(jnp.int32, (R, R), 1) // NB
    same = ri == ci

    def w_k(mu_ref, sg_ref, k):
        d0 = (rho - mu_ref[k, 0]) / (1e-14 + sg_ref[k, 0])
        d1 = (theta - mu_ref[k, 1]) / (1e-14 + sg_ref[k, 1])
        return jnp.where(same, jnp.exp(-0.5 * (d0 * d0 + d1 * d1)), 0.0)

    def gconv(mu_ref, sg_ref, W_ref, b_ref, src, dst, dst_off):
        Do = W_ref.shape[2]
        for k in range(_NK):
            agg = lax.dot_general(w_k(mu_ref, sg_ref, k), src,
                                  (((1,), (0,)), ((), ())),
                                  preferred_element_type=jnp.float32)
            o = lax.dot_general(agg, W_ref[k], (((1,), (0,)), ((), ())),
                                preferred_element_type=jnp.float32)
            dst[:, dst_off + k * Do:dst_off + (k + 1) * Do] = jax.nn.relu(
                o + b_ref[:, k * Do:(k + 1) * Do])

    gconv(g1mu_ref, g1sg_ref, g1W_ref, g1b_ref, X, hg1_s, 0)
    HG1 = hg1_s[...]
    gf_s[:, :F] = X
    gconv(g2mu_ref, g2sg_ref, g2W_ref, g2b_ref, HG1, gf_s, F)
    GF = gf_s[...]
    PROJ = jax.nn.relu(
        lax.dot_general(GF, gaW_ref[...], (((1,), (1,)), ((), ())),
                        preferred_element_type=jnp.float32) + gab_ref[...])
    qv = q_ref[...]
    Q = jnp.reshape(jnp.broadcast_to(qv[:, None, :], (S, NB, H)), (R, H))
    rawv = jnp.sum(PROJ * (Q * gattW_ref[...]), axis=1, keepdims=True)
    rawv = rawv + gattb_ref[0, 0]
    Rm = jnp.reshape(rawv, (S, NB))
    m = jnp.max(Rm, axis=1, keepdims=True)
    e = jnp.exp(Rm - m)
    A = e / jnp.sum(e, axis=1, keepdims=True)

    arow = jnp.reshape(A, (1, R))
    si = lax.broadcasted_iota(jnp.int32, (S, R), 0)
    cj = lax.broadcasted_iota(jnp.int32, (S, R), 1) // NB
    ATT = jnp.where(si == cj, jnp.broadcast_to(arow, (S, R)), 0.0)
    ATTD = lax.dot_general(ATT, GF, (((1,), (0,)), ((), ())),
                           preferred_element_type=jnp.float32)
    C = jax.nn.relu(
        lax.dot_general(ATTD, gaW_ref[...], (((1,), (1,)), ((), ())),
                        preferred_element_type=jnp.float32) + gab_ref[...])
    out_ref[...] = C * qv


def _graph_stage(topk_img, q_ga, gc1_mu, gc1_sigma, gc1_W, gc1_b,
                 gc2_mu, gc2_sigma, gc2_W, gc2_b,
                 ga_img_W, ga_img_b, ga_att_W, ga_att_b):
    B, NB, F = topk_img.shape
    H = q_ga.shape[1]
    S = _S_G
    full = lambda shape: pl.BlockSpec(shape, lambda i: (0,) * len(shape))
    smem = lambda shape: pl.BlockSpec(memory_space=pltpu.SMEM)
    return pl.pallas_call(
        _graph_body,
        grid=(B // S,),
        in_specs=[
            pl.BlockSpec((S, NB, F), lambda i: (i, 0, 0)),
            pl.BlockSpec((S, H), lambda i: (i, 0)),
            smem(gc1_mu.shape), smem(gc1_sigma.shape),
            full(gc1_W.shape), full((1, 2 * H)),
            smem(gc2_mu.shape), smem(gc2_sigma.shape),
            full(gc2_W.shape), full((1, H)),
            full(ga_img_W.shape), full((1, H)), full((1, H)), full((1, 1)),
        ],
        out_specs=pl.BlockSpec((S, H), lambda i: (i, 0)),
        out_shape=jax.ShapeDtypeStruct((B, H), jnp.float32),
        scratch_shapes=[
            pltpu.VMEM((S * NB, 2 * H), jnp.float32),
            pltpu.VMEM((S * NB, F + H), jnp.float32),
        ],
    )(topk_img, q_ga, gc1_mu, gc1_sigma, gc1_W, gc1_b.reshape(1, -1),
      gc2_mu, gc2_sigma, gc2_W, gc2_b.reshape(1, -1),
      ga_img_W, ga_img_b.reshape(1, -1), ga_att_W.reshape(1, -1),
      ga_att_b.reshape(1, 1))


_T_OUT = 512


def _head1_body(c2_ref, c1_ref, o1W_ref, o1b_ref, io1W_ref, io1b_ref,
                h1_ref, h2_ref):
    h1_ref[...] = jax.nn.relu(
        lax.dot_general(c2_ref[...], o1W_ref[...], (((1,), (1,)), ((), ())),
                        preferred_element_type=jnp.float32) + o1b_ref[...])
    h2_ref[...] = jax.nn.relu(
        lax.dot_general(c1_ref[...], io1W_ref[...], (((1,), (1,)), ((), ())),
                        preferred_element_type=jnp.float32) + io1b_ref[...])


def _head2_body(h1_ref, h2_ref, o2W_ref, o2b_ref, io2W_ref, io2b_ref,
                out_ref):
    out_ref[...] = (
        lax.dot_general(h1_ref[...], o2W_ref[...], (((1,), (1,)), ((), ())),
                        preferred_element_type=jnp.float32)
        + lax.dot_general(h2_ref[...], io2W_ref[...], (((1,), (1,)), ((), ())),
                          preferred_element_type=jnp.float32)
        + o2b_ref[...] + io2b_ref[...])


def _heads(comb2, comb1, out1_W, out1_b, out2_W, out2_b,
           iout1_W, iout1_b, iout2_W, iout2_b):
    B, H = comb2.shape
    O = out1_W.shape[0]
    T = _T_OUT
    nj = pl.cdiv(O, T)
    full = lambda shape: pl.BlockSpec(shape, lambda j: (0,) * len(shape))
    h1, h2 = pl.pallas_call(
        _head1_body,
        grid=(nj,),
        in_specs=[
            full((B, H)), full((B, H)),
            pl.BlockSpec((T, H), lambda j: (j, 0)),
            pl.BlockSpec((1, T), lambda j: (0, j)),
            pl.BlockSpec((T, H), lambda j: (j, 0)),
            pl.BlockSpec((1, T), lambda j: (0, j)),
        ],
        out_specs=[pl.BlockSpec((B, T), lambda j: (0, j))] * 2,
        out_shape=[jax.ShapeDtypeStruct((B, O), jnp.float32)] * 2,
    )(comb2, comb1, out1_W, out1_b.reshape(1, -1),
      iout1_W, iout1_b.reshape(1, -1))

    return pl.pallas_call(
        _head2_body,
        grid=(nj,),
        in_specs=[
            full((B, O)), full((B, O)),
            pl.BlockSpec((T, O), lambda j: (j, 0)),
            pl.BlockSpec((1, T), lambda j: (0, j)),
            pl.BlockSpec((T, O), lambda j: (j, 0)),
            pl.BlockSpec((1, T), lambda j: (0, j)),
        ],
        out_specs=pl.BlockSpec((B, T), lambda j: (0, j)),
        out_shape=jax.ShapeDtypeStruct((B, O), jnp.float32),
    )(h1, h2, out2_W, out2_b.reshape(1, -1), iout2_W, iout2_b.reshape(1, -1))


def kernel(question, image, K, qlen, wembed, W_ih, W_hh, b_ih, b_hh,
           ia_img_W, ia_img_b, ia_txt_W, ia_txt_b, ia_att_W, ia_att_b,
           ga_img_W, ga_img_b, ga_txt_W, ga_txt_b, ga_att_W, ga_att_b,
           gc1_mu, gc1_sigma, gc1_W, gc1_b, gc2_mu, gc2_sigma, gc2_W, gc2_b,
           out1_W, out1_b, out2_W, out2_b, iout1_W, iout1_b, iout2_W, iout2_b):
    B, T = question.shape
    V, E = wembed.shape

    EP = ((E + 127) // 128) * 128
    table = _pad_cols(wembed, EP) if EP != E else wembed
    idx = question.T.reshape(-1).astype(jnp.int32)
    emb = _embed_rows(table, idx).reshape(T, B, EP)

    W_ih_p = jnp.pad(W_ih, ((0, 0), (0, EP - E))) if EP != E else W_ih
    q_ia, q_ga = _gru_encode(emb, W_ih_p, W_hh, b_ih, b_hh,
                             ia_txt_W, ia_txt_b, ga_txt_W, ga_txt_b)

    comb1, topk_img = _attention1(image, q_ia, ia_img_W, ia_img_b,
                                  ia_att_W, ia_att_b)

    comb2 = _graph_stage(topk_img, q_ga, gc1_mu, gc1_sigma, gc1_W, gc1_b,
                         gc2_mu, gc2_sigma, gc2_W, gc2_b,
                         ga_img_W, ga_img_b, ga_att_W, ga_att_b)

    return _heads(comb2, comb1, out1_W, out1_b, out2_W, out2_b,
                  iout1_W, iout1_b, iout2_W, iout2_b)


# speedup vs baseline: 1.3547x; 1.0232x over previous
import functools

import jax
import jax.numpy as jnp
from jax import lax
from jax.experimental import pallas as pl
from jax.experimental.pallas import tpu as pltpu
from jax.experimental.pallas import tpu_sc as plsc


def _embed_rows(table, idx):
    V, D = table.shape
    N = idx.shape[0]
    NC, NS = 2, 16
    NW = NC * NS
    bpw = N // NW
    mesh = plsc.VectorSubcoreMesh(core_axis_name="c", subcore_axis_name="s")

    @functools.partial(
        pl.kernel,
        mesh=mesh,
        out_type=jax.ShapeDtypeStruct((N, D), jnp.float32),
        scratch_types=[
            pltpu.VMEM((bpw,), jnp.int32),
            pltpu.VMEM((bpw, D), jnp.float32),
            pltpu.SemaphoreType.DMA,
        ],
    )
    def k(table_hbm, idx_hbm, out_hbm, idx_v, rows_v, sem):
        wid = lax.axis_index("s") * NC + lax.axis_index("c")
        base = wid * bpw
        pltpu.sync_copy(idx_hbm.at[pl.ds(base, bpw)], idx_v)
        pltpu.async_copy(table_hbm.at[idx_v], rows_v, sem).wait()
        pltpu.sync_copy(rows_v, out_hbm.at[pl.ds(base, bpw)])

    return k(table, idx)


def _pad_body(src_ref, dst_ref):
    E = src_ref.shape[1]
    dst_ref[...] = jnp.zeros_like(dst_ref)
    dst_ref[:, :E] = src_ref[...]


def _pad_cols(src, EP):
    V, E = src.shape
    TV = 2000
    return pl.pallas_call(
        _pad_body,
        grid=(V // TV,),
        in_specs=[pl.BlockSpec((TV, E), lambda i: (i, 0))],
        out_specs=pl.BlockSpec((TV, EP), lambda i: (i, 0)),
        out_shape=jax.ShapeDtypeStruct((V, EP), jnp.float32),
    )(src)


def _gru_body(emb_ref, wih_ref, whh_ref, bih_ref, bhh_ref,
              iatW_ref, iatb_ref, gatW_ref, gatb_ref,
              qia_ref, qga_ref, h_scr):
    t = pl.program_id(0)
    nt = pl.num_programs(0)

    @pl.when(t == 0)
    def _():
        h_scr[...] = jnp.zeros_like(h_scr)

    x = emb_ref[0]
    h = h_scr[...]
    H = h.shape[1]
    gi = lax.dot_general(x, wih_ref[...], (((1,), (1,)), ((), ())),
                         preferred_element_type=jnp.float32) + bih_ref[...]
    gh = lax.dot_general(h, whh_ref[...], (((1,), (1,)), ((), ())),
                         preferred_element_type=jnp.float32) + bhh_ref[...]
    r = jax.nn.sigmoid(gi[:, :H] + gh[:, :H])
    z = jax.nn.sigmoid(gi[:, H:2 * H] + gh[:, H:2 * H])
    n = jnp.tanh(gi[:, 2 * H:] + r * gh[:, 2 * H:])
    h_new = (1.0 - z) * n + z * h
    h_scr[...] = h_new

    @pl.when(t == nt - 1)
    def _():
        qia_ref[...] = jax.nn.relu(
            lax.dot_general(h_new, iatW_ref[...], (((1,), (1,)), ((), ())),
                            preferred_element_type=jnp.float32) + iatb_ref[...])
        qga_ref[...] = jax.nn.relu(
            lax.dot_general(h_new, gatW_ref[...], (((1,), (1,)), ((), ())),
                            preferred_element_type=jnp.float32) + gatb_ref[...])


def _gru_encode(emb_tbe, W_ih, W_hh, b_ih, b_hh, ia_txt_W, ia_txt_b,
                ga_txt_W, ga_txt_b):
    T, B, EP = emb_tbe.shape
    H = W_hh.shape[1]
    full = lambda shape: pl.BlockSpec(shape, lambda t: (0,) * len(shape))
    return pl.pallas_call(
        _gru_body,
        grid=(T,),
        in_specs=[
            pl.BlockSpec((1, B, EP), lambda t: (t, 0, 0)),
            full(W_ih.shape), full(W_hh.shape),
            full((1, 3 * H)), full((1, 3 * H)),
            full(ia_txt_W.shape), full((1, H)),
            full(ga_txt_W.shape), full((1, H)),
        ],
        out_specs=[full((B, H)), full((B, H))],
        out_shape=[jax.ShapeDtypeStruct((B, H), jnp.float32)] * 2,
        scratch_shapes=[pltpu.VMEM((B, H), jnp.float32)],
    )(emb_tbe, W_ih, W_hh, b_ih.reshape(1, -1), b_hh.reshape(1, -1),
      ia_txt_W, ia_txt_b.reshape(1, -1), ga_txt_W, ga_txt_b.reshape(1, -1))


_S_ATT = 8
_NBH = 10


def _att_body(img_ref, q_ref, iaW_ref, iab_ref, attW_ref, attb_ref,
              comb_ref, topk_ref):
    S, L, F = img_ref.shape
    R = S * L
    H = q_ref.shape[2]
    X = jnp.reshape(img_ref[...], (R, F))
    PROJ = jax.nn.relu(
        lax.dot_general(X, iaW_ref[...], (((1,), (1,)), ((), ())),
                        preferred_element_type=jnp.float32) + iab_ref[...])
    qv = q_ref[0]
    Q = jnp.reshape(jnp.broadcast_to(qv[:, None, :], (S, L, H)), (R, H))
    joint = PROJ * Q
    jb = joint.astype(jnp.bfloat16).astype(jnp.float32)
    ab = attW_ref[...].astype(jnp.bfloat16).astype(jnp.float32)
    rawc = jnp.sum(jb * ab, axis=1, keepdims=True)
    raw = jnp.reshape(rawc, (S, L)) + attb_ref[0, 0]
    m = jnp.max(raw, axis=1, keepdims=True)
    e = jnp.exp(raw - m)
    att = e / jnp.sum(e, axis=1, keepdims=True)

    ri = lax.broadcasted_iota(jnp.int32, (S, R), 0)
    cj = lax.broadcasted_iota(jnp.int32, (S, R), 1) // L
    ATT = jnp.where(ri == cj, jnp.broadcast_to(jnp.reshape(att, (1, R)),
                                               (S, R)), 0.0)
    IMGATT = lax.dot_general(ATT, X, (((1,), (0,)), ((), ())),
                             preferred_element_type=jnp.float32)
    C = jax.nn.relu(
        lax.dot_general(IMGATT, iaW_ref[...], (((1,), (1,)), ((), ())),
                        preferred_element_type=jnp.float32) + iab_ref[...])
    comb_ref[0] = C * qv

    G = S * _NBH
    scoresG = jnp.reshape(jnp.broadcast_to(raw[:, None, :], (S, _NBH, L)),
                          (G, L))
    iotaG = lax.broadcasted_iota(jnp.int32, (G, L), 1)
    rq = lax.broadcasted_iota(jnp.int32, (G, 1), 0)
    sG = rq // _NBH
    jG = rq % _NBH
    tgt = jnp.zeros((G, 1), jnp.int32)
    for j in range(_NBH):
        mv = jnp.max(scoresG, axis=1, keepdims=True)
        idx = jnp.min(jnp.where(scoresG == mv, iotaG, L),
                      axis=1, keepdims=True)
        tgt = jnp.where(jG == j, idx + L * sG, tgt)
        scoresG = jnp.where(iotaG == idx, -jnp.inf, scoresG)

    colI = lax.broadcasted_iota(jnp.int32, (G, R), 1)
    SEL = (colI == tgt).astype(jnp.bfloat16)
    hi = X.astype(jnp.bfloat16)
    mid = (X - hi.astype(jnp.float32)).astype(jnp.bfloat16)
    lo = (X - hi.astype(jnp.float32)
          - mid.astype(jnp.float32)).astype(jnp.bfloat16)
    dn = (((1,), (0,)), ((), ()))
    TK = (lax.dot_general(SEL, hi, dn, preferred_element_type=jnp.float32)
          + lax.dot_general(SEL, mid, dn, preferred_element_type=jnp.float32)
          + lax.dot_general(SEL, lo, dn, preferred_element_type=jnp.float32))
    topk_ref[...] = jnp.reshape(TK, (S, _NBH, F))


def _attention1(image, q_ia, ia_img_W, ia_img_b, ia_att_W, ia_att_b):
    B, L, F = image.shape
    H = ia_img_W.shape[0]
    S = _S_ATT
    full = lambda shape: pl.BlockSpec(shape, lambda i: (0,) * len(shape))
    comb3, topk = pl.pallas_call(
        _att_body,
        grid=(B // S,),
        in_specs=[
            pl.BlockSpec((S, L, F), lambda i: (i, 0, 0)),
            pl.BlockSpec((1, S, H), lambda i: (i, 0, 0)),
            full(ia_img_W.shape), full((1, H)), full((1, H)), full((1, 1)),
        ],
        out_specs=[
            pl.BlockSpec((1, S, H), lambda i: (i, 0, 0)),
            pl.BlockSpec((S, _NBH, F), lambda i: (i, 0, 0)),
        ],
        out_shape=[
            jax.ShapeDtypeStruct((B // S, S, H), jnp.float32),
            jax.ShapeDtypeStruct((B, _NBH, F), jnp.float32),
        ],
    )(image, q_ia.reshape(B // S, S, H), ia_img_W, ia_img_b.reshape(1, -1),
      ia_att_W.reshape(1, -1), ia_att_b.reshape(1, 1))
    return comb3.reshape(B, H), topk


_S_G = 16
_NK = 8


def _graph_body(ti_ref, q_ref, g1mu_ref, g1sg_ref, g1W_ref, g1b_ref,
                g2mu_ref, g2sg_ref, g2W_ref, g2b_ref,
                gaW_ref, gab_ref, gattW_ref, gattb_ref, out_ref,
                hg1_s, gf_s):
    S, NB, F = ti_ref.shape
    R = S * NB
    X = jnp.reshape(ti_ref[...], (R, F))
    H2 = g1W_ref.shape[2] * _NK
    H = g2W_ref.shape[2] * _NK

    bb = X[:, F - 4:]
    cx = bb[:, 0:1] + 0.5 * (bb[:, 2:3] - bb[:, 0:1])
    cy = bb[:, 1:2] + 0.5 * (bb[:, 3:4] - bb[:, 1:2])
    pcx = cx - jnp.transpose(cx)
    pcy = cy - jnp.transpose(cy)
    rho = jnp.sqrt(pcx * pcx + pcy * pcy)
    theta = jnp.arctan2(pcx, pcy)

    ri = lax.broadcasted_iota(jnp.int32, (R, R), 0) // NB
    ci = lax.broadcasted_iota(jnp.int32, (R, R), 1) // NB
    same = ri == ci

    def w_k(mu_ref, sg_ref, k):
        d0 = (rho - mu_ref[k, 0]) / (1e-14 + sg_ref[k, 0])
        d1 = (theta - mu_ref[k, 1]) / (1e-14 + sg_ref[k, 1])
        return jnp.where(same, jnp.exp(-0.5 * (d0 * d0 + d1 * d1)), 0.0)

    def gconv(mu_ref, sg_ref, W_ref, b_ref, src, dst, dst_off):
        Do = W_ref.shape[2]
        for k in range(_NK):
            agg = lax.dot_general(w_k(mu_ref, sg_ref, k), src,
                                  (((1,), (0,)), ((), ())),
                                  preferred_element_type=jnp.float32)
            o = lax.dot_general(agg, W_ref[k], (((1,), (0,)), ((), ())),
                                preferred_element_type=jnp.float32)
            dst[:, dst_off + k * Do:dst_off + (k + 1) * Do] = jax.nn.relu(
                o + b_ref[:, k * Do:(k + 1) * Do])

    gconv(g1mu_ref, g1sg_ref, g1W_ref, g1b_ref, X, hg1_s, 0)
    HG1 = hg1_s[...]
    gf_s[:, :F] = X
    gconv(g2mu_ref, g2sg_ref, g2W_ref, g2b_ref, HG1, gf_s, F)
    GF = gf_s[...]
    PROJ = jax.nn.relu(
        lax.dot_general(GF, gaW_ref[...], (((1,), (1,)), ((), ())),
                        preferred_element_type=jnp.float32) + gab_ref[...])
    qv = q_ref[...]
    Q = jnp.reshape(jnp.broadcast_to(qv[:, None, :], (S, NB, H)), (R, H))
    rawv = jnp.sum(PROJ * (Q * gattW_ref[...]), axis=1, keepdims=True)
    rawv = rawv + gattb_ref[0, 0]
    Rm = jnp.reshape(rawv, (S, NB))
    m = jnp.max(Rm, axis=1, keepdims=True)
    e = jnp.exp(Rm - m)
    A = e / jnp.sum(e, axis=1, keepdims=True)

    arow = jnp.reshape(A, (1, R))
    si = lax.broadcasted_iota(jnp.int32, (S, R), 0)
    cj = lax.broadcasted_iota(jnp.int32, (S, R), 1) // NB
    ATT = jnp.where(si == cj, jnp.broadcast_to(arow, (S, R)), 0.0)
    ATTD = lax.dot_general(ATT, GF, (((1,), (0,)), ((), ())),
                           preferred_element_type=jnp.float32)
    C = jax.nn.relu(
        lax.dot_general(ATTD, gaW_ref[...], (((1,), (1,)), ((), ())),
                        preferred_element_type=jnp.float32) + gab_ref[...])
    out_ref[...] = C * qv


def _graph_stage(topk_img, q_ga, gc1_mu, gc1_sigma, gc1_W, gc1_b,
                 gc2_mu, gc2_sigma, gc2_W, gc2_b,
                 ga_img_W, ga_img_b, ga_att_W, ga_att_b):
    B, NB, F = topk_img.shape
    H = q_ga.shape[1]
    S = _S_G
    full = lambda shape: pl.BlockSpec(shape, lambda i: (0,) * len(shape))
    smem = lambda shape: pl.BlockSpec(memory_space=pltpu.SMEM)
    return pl.pallas_call(
        _graph_body,
        grid=(B // S,),
        in_specs=[
            pl.BlockSpec((S, NB, F), lambda i: (i, 0, 0)),
            pl.BlockSpec((S, H), lambda i: (i, 0)),
            smem(gc1_mu.shape), smem(gc1_sigma.shape),
            full(gc1_W.shape), full((1, 2 * H)),
            smem(gc2_mu.shape), smem(gc2_sigma.shape),
            full(gc2_W.shape), full((1, H)),
            full(ga_img_W.shape), full((1, H)), full((1, H)), full((1, 1)),
        ],
        out_specs=pl.BlockSpec((S, H), lambda i: (i, 0)),
        out_shape=jax.ShapeDtypeStruct((B, H), jnp.float32),
        scratch_shapes=[
            pltpu.VMEM((S * NB, 2 * H), jnp.float32),
            pltpu.VMEM((S * NB, F + H), jnp.float32),
        ],
    )(topk_img, q_ga, gc1_mu, gc1_sigma, gc1_W, gc1_b.reshape(1, -1),
      gc2_mu, gc2_sigma, gc2_W, gc2_b.reshape(1, -1),
      ga_img_W, ga_img_b.reshape(1, -1), ga_att_W.reshape(1, -1),
      ga_att_b.reshape(1, 1))


_T_OUT = 512


def _head1_body(c2_ref, c1_ref, o1W_ref, o1b_ref, io1W_ref, io1b_ref,
                h1_ref, h2_ref):
    h1_ref[...] = jax.nn.relu(
        lax.dot_general(c2_ref[...], o1W_ref[...], (((1,), (1,)), ((), ())),
                        preferred_element_type=jnp.float32) + o1b_ref[...])
    h2_ref[...] = jax.nn.relu(
        lax.dot_general(c1_ref[...], io1W_ref[...], (((1,), (1,)), ((), ())),
                        preferred_element_type=jnp.float32) + io1b_ref[...])


def _head2_body(h1_ref, h2_ref, o2W_ref, o2b_ref, io2W_ref, io2b_ref,
                out_ref):
    out_ref[...] = (
        lax.dot_general(h1_ref[...], o2W_ref[...], (((1,), (1,)), ((), ())),
                        preferred_element_type=jnp.float32)
        + lax.dot_general(h2_ref[...], io2W_ref[...], (((1,), (1,)), ((), ())),
                          preferred_element_type=jnp.float32)
        + o2b_ref[...] + io2b_ref[...])


def _heads(comb2, comb1, out1_W, out1_b, out2_W, out2_b,
           iout1_W, iout1_b, iout2_W, iout2_b):
    B, H = comb2.shape
    O = out1_W.shape[0]
    T = _T_OUT
    nj = pl.cdiv(O, T)
    full = lambda shape: pl.BlockSpec(shape, lambda j: (0,) * len(shape))
    h1, h2 = pl.pallas_call(
        _head1_body,
        grid=(nj,),
        in_specs=[
            full((B, H)), full((B, H)),
            pl.BlockSpec((T, H), lambda j: (j, 0)),
            pl.BlockSpec((1, T), lambda j: (0, j)),
            pl.BlockSpec((T, H), lambda j: (j, 0)),
            pl.BlockSpec((1, T), lambda j: (0, j)),
        ],
        out_specs=[pl.BlockSpec((B, T), lambda j: (0, j))] * 2,
        out_shape=[jax.ShapeDtypeStruct((B, O), jnp.float32)] * 2,
    )(comb2, comb1, out1_W, out1_b.reshape(1, -1),
      iout1_W, iout1_b.reshape(1, -1))

    return pl.pallas_call(
        _head2_body,
        grid=(nj,),
        in_specs=[
            full((B, O)), full((B, O)),
            pl.BlockSpec((T, O), lambda j: (j, 0)),
            pl.BlockSpec((1, T), lambda j: (0, j)),
            pl.BlockSpec((T, O), lambda j: (j, 0)),
            pl.BlockSpec((1, T), lambda j: (0, j)),
        ],
        out_specs=pl.BlockSpec((B, T), lambda j: (0, j)),
        out_shape=jax.ShapeDtypeStruct((B, O), jnp.float32),
    )(h1, h2, out2_W, out2_b.reshape(1, -1), iout2_W, iout2_b.reshape(1, -1))


def kernel(question, image, K, qlen, wembed, W_ih, W_hh, b_ih, b_hh,
           ia_img_W, ia_img_b, ia_txt_W, ia_txt_b, ia_att_W, ia_att_b,
           ga_img_W, ga_img_b, ga_txt_W, ga_txt_b, ga_att_W, ga_att_b,
           gc1_mu, gc1_sigma, gc1_W, gc1_b, gc2_mu, gc2_sigma, gc2_W, gc2_b,
           out1_W, out1_b, out2_W, out2_b, iout1_W, iout1_b, iout2_W, iout2_b):
    B, T = question.shape
    V, E = wembed.shape

    EP = ((E + 127) // 128) * 128
    table = _pad_cols(wembed, EP) if EP != E else wembed
    idx = question.T.reshape(-1).astype(jnp.int32)
    emb = _embed_rows(table, idx).reshape(T, B, EP)

    W_ih_p = jnp.pad(W_ih, ((0, 0), (0, EP - E))) if EP != E else W_ih
    q_ia, q_ga = _gru_encode(emb, W_ih_p, W_hh, b_ih, b_hh,
                             ia_txt_W, ia_txt_b, ga_txt_W, ga_txt_b)

    comb1, topk_img = _attention1(image, q_ia, ia_img_W, ia_img_b,
                                  ia_att_W, ia_att_b)

    comb2 = _graph_stage(topk_img, q_ga, gc1_mu, gc1_sigma, gc1_W, gc1_b,
                         gc2_mu, gc2_sigma, gc2_W, gc2_b,
                         ga_img_W, ga_img_b, ga_att_W, ga_att_b)

    return _heads(comb2, comb1, out1_W, out1_b, out2_W, out2_b,
                  iout1_W, iout1_b, iout2_W, iout2_b)
